# baseline (device time: 13570 ns/iter reference)
import jax
import jax.numpy as jnp
from jax import lax
from jax.experimental import pallas as pl
from jax.experimental.pallas import tpu as pltpu

N_DEV = 8
T = 256
V_LOCAL = 4096


def kernel(x, W, labels):
    labels2 = labels.reshape(1, T).astype(jnp.int32)

    def body(x_ref, w_ref, labels_ref, out_ref, comm_ref, send_sems, recv_sems):
        my = lax.axis_index("i")

        barrier = pltpu.get_barrier_semaphore()
        for d in range(1, N_DEV):
            pl.semaphore_signal(
                barrier,
                inc=1,
                device_id=((my + d) % N_DEV,),
                device_id_type=pl.DeviceIdType.MESH,
            )
        xb = x_ref[:, :].astype(jnp.bfloat16)
        wb = w_ref[:, :].astype(jnp.bfloat16)
        logits_t = lax.dot_general(
            wb,
            xb,
            (((0,), (1,)), ((), ())),
            preferred_element_type=jnp.float32,
        )

        s = jnp.sum(jnp.exp(logits_t), axis=0, keepdims=True)

        vids = lax.broadcasted_iota(jnp.int32, (V_LOCAL, T), 0) + my * V_LOCAL
        mask = vids == labels_ref[:, :]
        ll = jnp.sum(jnp.where(mask, logits_t, 0.0), axis=0, keepdims=True)

        comm_ref[0, :, :] = jnp.concatenate([s, ll], axis=0)

        pl.semaphore_wait(barrier, N_DEV - 1)

        sends = []
        for d in (4, 3, 5, 2, 6, 1, 7):
            rdma = pltpu.make_async_remote_copy(
                src_ref=comm_ref.at[0],
                dst_ref=comm_ref.at[d],
                send_sem=send_sems.at[d],
                recv_sem=recv_sems.at[d],
                device_id=((my + d) % N_DEV,),
                device_id_type=pl.DeviceIdType.MESH,
            )
            rdma.start()
            sends.append(rdma)

        for rdma in sends:
            rdma.wait_recv()
        for rdma in sends:
            rdma.wait_send()

        c = comm_ref[:, :, :]
        s_g = jnp.sum(c[:, 0:1, :], axis=0)
        ll_g = jnp.sum(c[:, 1:2, :], axis=0)
        out_ref[:, :] = jnp.log(s_g) - ll_g

    out = pl.pallas_call(
        body,
        out_shape=jax.ShapeDtypeStruct((1, T), jnp.float32),
        in_specs=[
            pl.BlockSpec(memory_space=pltpu.VMEM),
            pl.BlockSpec(memory_space=pltpu.VMEM),
            pl.BlockSpec(memory_space=pltpu.VMEM),
        ],
        out_specs=pl.BlockSpec(memory_space=pltpu.VMEM),
        scratch_shapes=[
            pltpu.VMEM((N_DEV, 2, T), jnp.float32),
            pltpu.SemaphoreType.DMA((N_DEV,)),
            pltpu.SemaphoreType.DMA((N_DEV,)),
        ],
        compiler_params=pltpu.CompilerParams(collective_id=0),
    )(x, W, labels2)
    return out.reshape(T)


# device time: 13527 ns/iter; 1.0032x vs baseline; 1.0032x over previous
import jax
import jax.numpy as jnp
from jax import lax
from jax.experimental import pallas as pl
from jax.experimental.pallas import tpu as pltpu

N_DEV = 8
T = 256
V_LOCAL = 4096


def kernel(x, W, labels):

    def body(x_ref, w_ref, labels_ref, out_ref, comm_ref, send_sems, recv_sems):
        my = lax.axis_index("i")

        barrier = pltpu.get_barrier_semaphore()
        for d in range(1, N_DEV):
            pl.semaphore_signal(
                barrier,
                inc=1,
                device_id=((my + d) % N_DEV,),
                device_id_type=pl.DeviceIdType.MESH,
            )
        xb = x_ref[:, :].astype(jnp.bfloat16)
        wb = w_ref[:, :].astype(jnp.bfloat16)
        logits_t = lax.dot_general(
            wb,
            xb,
            (((0,), (1,)), ((), ())),
            preferred_element_type=jnp.float32,
        )

        s = jnp.sum(jnp.exp(logits_t), axis=0, keepdims=True)

        vids = lax.broadcasted_iota(jnp.int32, (V_LOCAL, T), 0) + my * V_LOCAL
        mask = vids == labels_ref[:].reshape(1, T)
        ll = jnp.sum(jnp.where(mask, logits_t, 0.0), axis=0, keepdims=True)

        comm_ref[0, :, :] = jnp.concatenate([s, ll], axis=0)

        pl.semaphore_wait(barrier, N_DEV - 1)

        sends = []
        for d in (4, 3, 5, 2, 6, 1, 7):
            rdma = pltpu.make_async_remote_copy(
                src_ref=comm_ref.at[0],
                dst_ref=comm_ref.at[d],
                send_sem=send_sems.at[d],
                recv_sem=recv_sems.at[d],
                device_id=((my + d) % N_DEV,),
                device_id_type=pl.DeviceIdType.MESH,
            )
            rdma.start()
            sends.append(rdma)

        for rdma in sends:
            rdma.wait_recv()
        for rdma in sends:
            rdma.wait_send()

        c = comm_ref[:, :, :]
        s_g = jnp.sum(c[:, 0:1, :], axis=0)
        ll_g = jnp.sum(c[:, 1:2, :], axis=0)
        out_ref[:] = (jnp.log(s_g) - ll_g).reshape(T)

    return pl.pallas_call(
        body,
        out_shape=jax.ShapeDtypeStruct((T,), jnp.float32),
        in_specs=[
            pl.BlockSpec(memory_space=pltpu.VMEM),
            pl.BlockSpec(memory_space=pltpu.VMEM),
            pl.BlockSpec(memory_space=pltpu.VMEM),
        ],
        out_specs=pl.BlockSpec(memory_space=pltpu.VMEM),
        scratch_shapes=[
            pltpu.VMEM((N_DEV, 2, T), jnp.float32),
            pltpu.SemaphoreType.DMA((N_DEV,)),
            pltpu.SemaphoreType.DMA((N_DEV,)),
        ],
        compiler_params=pltpu.CompilerParams(collective_id=0),
    )(x, W, labels)
